# baseline (speedup 1.0000x reference)
import functools
import math

import jax
import jax.numpy as jnp
from jax import lax
from jax.experimental import pallas as pl
from jax.experimental.pallas import tpu as pltpu
from jax.experimental.pallas import tpu_sc as plsc

B, S, D, H, E = 2, 2048, 768, 1536, 64
T = B * S
BT = 128
NT = 96
XS_PAD = NT * BT
XH = D // 2
XW = XH + 128

NC, NS = 2, 16
NW = NC * NS
PER_W = T // NW
CH = 64


def _router_body(x_ref, wg_ref, bg_ref, pos_ref, pw_ref, eid_ref, blk_ref):
    x = x_ref[...]
    logits = jnp.dot(x, wg_ref[...], preferred_element_type=jnp.float32)
    logits = logits + bg_ref[...]
    m = jnp.max(logits, axis=1, keepdims=True)
    iota_e = lax.broadcasted_iota(jnp.int32, (T, E), 1)
    top1 = jnp.min(jnp.where(logits == m, iota_e, E), axis=1, keepdims=True)
    pmax = 1.0 / jnp.sum(jnp.exp(logits - m), axis=1, keepdims=True)

    onehot = (iota_e == top1).astype(jnp.int32)
    inc = onehot
    k = 1
    while k < T:
        shifted = jnp.concatenate(
            [jnp.zeros((k, E), jnp.int32), inc[: T - k, :]], axis=0)
        inc = inc + shifted
        k *= 2
    rank = jnp.sum(onehot * inc, axis=1, keepdims=True) - 1
    counts = inc[T - 1:T, :]

    ntiles = (counts + BT - 1) // BT
    cum = ntiles
    k = 1
    while k < E:
        shifted = jnp.concatenate(
            [jnp.zeros((1, k), jnp.int32), cum[:, : E - k]], axis=1)
        cum = cum + shifted
        k *= 2
    excl = cum - ntiles
    start_tok = jnp.sum(onehot * (BT * excl), axis=1, keepdims=True)
    pos_ref[...] = (start_tok + rank).reshape(T)

    x_bf = x.astype(jnp.bfloat16)
    lo = lax.bitcast_convert_type(x_bf[:, :XH], jnp.uint16).astype(jnp.uint32)
    hi = lax.bitcast_convert_type(x_bf[:, XH:], jnp.uint16).astype(jnp.uint32)
    xi = lo | (hi << 16)
    pb = lax.bitcast_convert_type(pmax.astype(jnp.bfloat16),
                                  jnp.uint16).astype(jnp.uint32)
    pi = jnp.broadcast_to(pb | (pb << 16), (T, XW - XH))
    pw_ref[...] = lax.bitcast_convert_type(
        jnp.concatenate([xi, pi], axis=1), jnp.int32)

    ii = lax.broadcasted_iota(jnp.int32, (NT, E), 0)
    cum_b = jnp.broadcast_to(cum, (NT, E))
    eid = jnp.sum((cum_b <= ii).astype(jnp.int32), axis=1, keepdims=True)
    iota_e_row = lax.broadcasted_iota(jnp.int32, (1, E), 1)
    last_e = jnp.max(jnp.where(counts > 0, iota_e_row, 0))
    eid_ref[...] = jnp.minimum(eid, last_e).reshape(NT)
    total = cum[0, E - 1]
    ii1 = lax.broadcasted_iota(jnp.int32, (NT, 1), 0)
    blk_ref[...] = jnp.where(ii1 < total, ii1, total - 1).reshape(NT)


def _run_router(xf, Wg, bg):
    return pl.pallas_call(
        _router_body,
        out_shape=[
            jax.ShapeDtypeStruct((T,), jnp.int32),
            jax.ShapeDtypeStruct((T, XW), jnp.int32),
            jax.ShapeDtypeStruct((NT,), jnp.int32),
            jax.ShapeDtypeStruct((NT,), jnp.int32),
        ],
    )(xf, Wg, bg.reshape(1, E))


def _dispatch_body(pos_hbm, pw_hbm, xs_hbm, idx_v, rows_v, sem_x):
    wid = lax.axis_index("s") * NC + lax.axis_index("c")
    base = wid * PER_W
    pltpu.sync_copy(pos_hbm.at[pl.ds(base, PER_W)], idx_v)
    pltpu.sync_copy(pw_hbm.at[pl.ds(base, PER_W)], rows_v)
    pltpu.async_copy(rows_v, xs_hbm.at[idx_v], sem_x).wait()


def _run_dispatch(pos, pw):
    f = functools.partial(
        pl.kernel,
        out_type=jax.ShapeDtypeStruct((XS_PAD, XW), jnp.int32),
        mesh=plsc.VectorSubcoreMesh(core_axis_name="c", subcore_axis_name="s"),
        scratch_types=[
            pltpu.VMEM((PER_W,), jnp.int32),
            pltpu.VMEM((PER_W, XW), jnp.int32),
            pltpu.SemaphoreType.DMA,
        ],
    )(_dispatch_body)
    return f(pos, pw)


def _ffn_body(eids, blks, xs_ref, w1a_ref, w1b_ref, b1_ref, w2a_ref, w2b_ref,
              b2_ref, os_ref):
    u = lax.bitcast_convert_type(xs_ref[...], jnp.uint32)
    xi = u[:, :XH]
    xlo = lax.bitcast_convert_type((xi & 0xFFFF).astype(jnp.uint16),
                                   jnp.bfloat16).astype(jnp.float32)
    xhi = lax.bitcast_convert_type((xi >> 16).astype(jnp.uint16),
                                   jnp.bfloat16).astype(jnp.float32)
    x = jnp.concatenate([xlo, xhi], axis=1)
    p = lax.bitcast_convert_type(
        (u[:, XH:XH + 1] & 0xFFFF).astype(jnp.uint16),
        jnp.bfloat16).astype(jnp.float32)
    h = jnp.concatenate(
        [jnp.dot(x, w1a_ref[0], preferred_element_type=jnp.float32),
         jnp.dot(x, w1b_ref[0], preferred_element_type=jnp.float32)], axis=1)
    h = h + b1_ref[0]
    h = 0.5 * h * (1.0 + lax.erf(h * (1.0 / math.sqrt(2.0))))
    o = (jnp.dot(h[:, :H // 2], w2a_ref[0],
                 preferred_element_type=jnp.float32)
         + jnp.dot(h[:, H // 2:], w2b_ref[0],
                   preferred_element_type=jnp.float32))
    o = o + b2_ref[0]
    os_ref[...] = o * p


def _run_ffn(eids, blks, xs, W1, b1, W2, b2):
    grid_spec = pltpu.PrefetchScalarGridSpec(
        num_scalar_prefetch=2,
        grid=(NT,),
        in_specs=[
            pl.BlockSpec((BT, XW), lambda i, eids, blks: (blks[i], 0)),
            pl.BlockSpec((1, D, H // 2), lambda i, eids, blks: (eids[i], 0, 0)),
            pl.BlockSpec((1, D, H // 2), lambda i, eids, blks: (eids[i], 0, 1)),
            pl.BlockSpec((1, 1, H), lambda i, eids, blks: (eids[i], 0, 0)),
            pl.BlockSpec((1, H // 2, D), lambda i, eids, blks: (eids[i], 0, 0)),
            pl.BlockSpec((1, H // 2, D), lambda i, eids, blks: (eids[i], 1, 0)),
            pl.BlockSpec((1, 1, D), lambda i, eids, blks: (eids[i], 0, 0)),
        ],
        out_specs=pl.BlockSpec((BT, D), lambda i, eids, blks: (blks[i], 0)),
    )
    return pl.pallas_call(
        _ffn_body,
        grid_spec=grid_spec,
        out_shape=jax.ShapeDtypeStruct((XS_PAD, D), jnp.float32),
    )(eids, blks, xs, W1, W1, b1.reshape(E, 1, H), W2, W2,
      b2.reshape(E, 1, D))


def _combine_body(pos_hbm, os_hbm, out_hbm, idx_v, rows_v, sem):
    wid = lax.axis_index("s") * NC + lax.axis_index("c")
    for c in range(PER_W // CH):
        base = wid * PER_W + c * CH
        pltpu.sync_copy(pos_hbm.at[pl.ds(base, CH)], idx_v)
        pltpu.async_copy(os_hbm.at[idx_v], rows_v, sem).wait()
        pltpu.sync_copy(rows_v, out_hbm.at[pl.ds(base, CH)])


def _run_combine(pos, os):
    f = functools.partial(
        pl.kernel,
        out_type=jax.ShapeDtypeStruct((T, D), jnp.float32),
        mesh=plsc.VectorSubcoreMesh(core_axis_name="c", subcore_axis_name="s"),
        scratch_types=[
            pltpu.VMEM((CH,), jnp.int32),
            pltpu.VMEM((CH, D), jnp.float32),
            pltpu.SemaphoreType.DMA,
        ],
    )(_combine_body)
    return f(pos, os)


@jax.jit
def kernel(x, W1, b1, W2, b2, Wg, bg):
    xf = x.reshape(T, D)
    pos, pw, eids, blks = _run_router(xf, Wg, bg)
    xs = _run_dispatch(pos, pw)
    os = _run_ffn(eids, blks, xs, W1, b1, W2, b2)
    out = _run_combine(pos, os)
    return out.reshape(B, S, D)


# speedup vs baseline: 1.0080x; 1.0080x over previous
import functools
import math

import jax
import jax.numpy as jnp
from jax import lax
from jax.experimental import pallas as pl
from jax.experimental.pallas import tpu as pltpu
from jax.experimental.pallas import tpu_sc as plsc

B, S, D, H, E = 2, 2048, 768, 1536, 64
T = B * S
BT = 128
NT = 96
XS_PAD = NT * BT
XH = D // 2
XW = XH + 128

NC, NS = 2, 16
NW = NC * NS
PER_W = T // NW
CH = 64


def _router_body(x_ref, wg_ref, bg_ref, pos_ref, pw_ref, eid_ref, blk_ref):
    x = x_ref[...]
    logits = jnp.dot(x, wg_ref[...], preferred_element_type=jnp.float32)
    logits = logits + bg_ref[...]
    m = jnp.max(logits, axis=1, keepdims=True)
    iota_e = lax.broadcasted_iota(jnp.int32, (T, E), 1)
    top1 = jnp.min(jnp.where(logits == m, iota_e, E), axis=1, keepdims=True)
    pmax = 1.0 / jnp.sum(jnp.exp(logits - m), axis=1, keepdims=True)

    onehot = (iota_e == top1).astype(jnp.int32)
    inc = onehot
    k = 1
    while k < T:
        shifted = jnp.concatenate(
            [jnp.zeros((k, E), jnp.int32), inc[: T - k, :]], axis=0)
        inc = inc + shifted
        k *= 2
    rank = jnp.sum(onehot * inc, axis=1, keepdims=True) - 1
    counts = inc[T - 1:T, :]

    ntiles = (counts + BT - 1) // BT
    cum = ntiles
    k = 1
    while k < E:
        shifted = jnp.concatenate(
            [jnp.zeros((1, k), jnp.int32), cum[:, : E - k]], axis=1)
        cum = cum + shifted
        k *= 2
    excl = cum - ntiles
    start_tok = jnp.sum(onehot * (BT * excl), axis=1, keepdims=True)
    pos_ref[...] = (start_tok + rank).reshape(T)

    x_bf = x.astype(jnp.bfloat16)
    lo = lax.bitcast_convert_type(x_bf[:, :XH], jnp.uint16).astype(jnp.uint32)
    hi = lax.bitcast_convert_type(x_bf[:, XH:], jnp.uint16).astype(jnp.uint32)
    xi = lo | (hi << 16)
    pb = lax.bitcast_convert_type(pmax.astype(jnp.bfloat16),
                                  jnp.uint16).astype(jnp.uint32)
    pi = jnp.broadcast_to(pb | (pb << 16), (T, XW - XH))
    pw_ref[...] = lax.bitcast_convert_type(
        jnp.concatenate([xi, pi], axis=1), jnp.int32)

    ii = lax.broadcasted_iota(jnp.int32, (NT, E), 0)
    cum_b = jnp.broadcast_to(cum, (NT, E))
    eid = jnp.sum((cum_b <= ii).astype(jnp.int32), axis=1, keepdims=True)
    iota_e_row = lax.broadcasted_iota(jnp.int32, (1, E), 1)
    last_e = jnp.max(jnp.where(counts > 0, iota_e_row, 0))
    eid_ref[...] = jnp.minimum(eid, last_e).reshape(NT)
    total = cum[0, E - 1]
    ii1 = lax.broadcasted_iota(jnp.int32, (NT, 1), 0)
    blk_ref[...] = jnp.where(ii1 < total, ii1, total - 1).reshape(NT)


def _run_router(xf, Wg, bg):
    return pl.pallas_call(
        _router_body,
        out_shape=[
            jax.ShapeDtypeStruct((T,), jnp.int32),
            jax.ShapeDtypeStruct((T, XW), jnp.int32),
            jax.ShapeDtypeStruct((NT,), jnp.int32),
            jax.ShapeDtypeStruct((NT,), jnp.int32),
        ],
    )(xf, Wg, bg.reshape(1, E))


def _dispatch_body(pos_hbm, pw_hbm, xs_hbm,
                   idx0, idx1, rows0, rows1, sem0, sem1):
    wid = lax.axis_index("s") * NC + lax.axis_index("c")
    base = wid * PER_W
    pltpu.sync_copy(pos_hbm.at[pl.ds(base, CH)], idx0)
    pltpu.sync_copy(pw_hbm.at[pl.ds(base, CH)], rows0)
    c0 = pltpu.async_copy(rows0, xs_hbm.at[idx0], sem0)
    pltpu.sync_copy(pos_hbm.at[pl.ds(base + CH, CH)], idx1)
    pltpu.sync_copy(pw_hbm.at[pl.ds(base + CH, CH)], rows1)
    c1 = pltpu.async_copy(rows1, xs_hbm.at[idx1], sem1)
    c0.wait()
    c1.wait()


def _run_dispatch(pos, pw):
    f = functools.partial(
        pl.kernel,
        out_type=jax.ShapeDtypeStruct((XS_PAD, XW), jnp.int32),
        mesh=plsc.VectorSubcoreMesh(core_axis_name="c", subcore_axis_name="s"),
        scratch_types=[
            pltpu.VMEM((CH,), jnp.int32),
            pltpu.VMEM((CH,), jnp.int32),
            pltpu.VMEM((CH, XW), jnp.int32),
            pltpu.VMEM((CH, XW), jnp.int32),
            pltpu.SemaphoreType.DMA,
            pltpu.SemaphoreType.DMA,
        ],
    )(_dispatch_body)
    return f(pos, pw)


def _ffn_body(eids, blks, xs_ref, w1_ref, b1_ref, w2_ref, b2_ref, os_ref):
    u = lax.bitcast_convert_type(xs_ref[...], jnp.uint32)
    xi = u[:, :XH]
    xlo = lax.bitcast_convert_type((xi & 0xFFFF).astype(jnp.uint16),
                                   jnp.bfloat16).astype(jnp.float32)
    xhi = lax.bitcast_convert_type((xi >> 16).astype(jnp.uint16),
                                   jnp.bfloat16).astype(jnp.float32)
    x = jnp.concatenate([xlo, xhi], axis=1)
    p = lax.bitcast_convert_type(
        (u[:, XH:XH + 1] & 0xFFFF).astype(jnp.uint16),
        jnp.bfloat16).astype(jnp.float32)
    h = jnp.dot(x, w1_ref[0], preferred_element_type=jnp.float32)
    h = h + b1_ref[0]
    h = 0.5 * h * (1.0 + lax.erf(h * (1.0 / math.sqrt(2.0))))
    o = jnp.dot(h, w2_ref[0], preferred_element_type=jnp.float32)
    o = o + b2_ref[0]
    os_ref[...] = o * p


def _run_ffn(eids, blks, xs, W1, b1, W2, b2):
    grid_spec = pltpu.PrefetchScalarGridSpec(
        num_scalar_prefetch=2,
        grid=(NT,),
        in_specs=[
            pl.BlockSpec((BT, XW), lambda i, eids, blks: (blks[i], 0)),
            pl.BlockSpec((1, D, H), lambda i, eids, blks: (eids[i], 0, 0)),
            pl.BlockSpec((1, 1, H), lambda i, eids, blks: (eids[i], 0, 0)),
            pl.BlockSpec((1, H, D), lambda i, eids, blks: (eids[i], 0, 0)),
            pl.BlockSpec((1, 1, D), lambda i, eids, blks: (eids[i], 0, 0)),
        ],
        out_specs=pl.BlockSpec((BT, D), lambda i, eids, blks: (blks[i], 0)),
    )
    return pl.pallas_call(
        _ffn_body,
        grid_spec=grid_spec,
        out_shape=jax.ShapeDtypeStruct((XS_PAD, D), jnp.float32),
    )(eids, blks, xs, W1, b1.reshape(E, 1, H), W2, b2.reshape(E, 1, D))


def _combine_body(pos_hbm, os_hbm, out_hbm,
                  idx0, idx1, rows0, rows1, sem0, sem1):
    wid = lax.axis_index("s") * NC + lax.axis_index("c")
    base = wid * PER_W
    pltpu.sync_copy(pos_hbm.at[pl.ds(base, CH)], idx0)
    g0 = pltpu.async_copy(os_hbm.at[idx0], rows0, sem0)
    pltpu.sync_copy(pos_hbm.at[pl.ds(base + CH, CH)], idx1)
    g1 = pltpu.async_copy(os_hbm.at[idx1], rows1, sem1)
    g0.wait()
    pltpu.sync_copy(rows0, out_hbm.at[pl.ds(base, CH)])
    g1.wait()
    pltpu.sync_copy(rows1, out_hbm.at[pl.ds(base + CH, CH)])


def _run_combine(pos, os):
    f = functools.partial(
        pl.kernel,
        out_type=jax.ShapeDtypeStruct((T, D), jnp.float32),
        mesh=plsc.VectorSubcoreMesh(core_axis_name="c", subcore_axis_name="s"),
        scratch_types=[
            pltpu.VMEM((CH,), jnp.int32),
            pltpu.VMEM((CH,), jnp.int32),
            pltpu.VMEM((CH, D), jnp.float32),
            pltpu.VMEM((CH, D), jnp.float32),
            pltpu.SemaphoreType.DMA,
            pltpu.SemaphoreType.DMA,
        ],
    )(_combine_body)
    return f(pos, os)


@jax.jit
def kernel(x, W1, b1, W2, b2, Wg, bg):
    xf = x.reshape(T, D)
    pos, pw, eids, blks = _run_router(xf, Wg, bg)
    xs = _run_dispatch(pos, pw)
    os = _run_ffn(eids, blks, xs, W1, b1, W2, b2)
    out = _run_combine(pos, os)
    return out.reshape(B, S, D)


# speedup vs baseline: 1.0161x; 1.0080x over previous
import functools
import math

import jax
import jax.numpy as jnp
from jax import lax
from jax.experimental import pallas as pl
from jax.experimental.pallas import tpu as pltpu
from jax.experimental.pallas import tpu_sc as plsc

B, S, D, H, E = 2, 2048, 768, 1536, 64
T = B * S
BT = 128
NT = 96
XS_PAD = NT * BT
XH = D // 2
XW = XH + 128

NC, NS = 2, 16
NW = NC * NS
PER_W = T // NW
CH = 64


def _router_body(x_ref, wg_ref, bg_ref, pos_ref, pw_ref, eid_ref, blk_ref):
    x = x_ref[...]
    logits = jnp.dot(x, wg_ref[...], preferred_element_type=jnp.float32)
    logits = logits + bg_ref[...]
    m = jnp.max(logits, axis=1, keepdims=True)
    iota_e = lax.broadcasted_iota(jnp.int32, (T, E), 1)
    top1 = jnp.min(jnp.where(logits == m, iota_e, E), axis=1, keepdims=True)
    pmax = 1.0 / jnp.sum(jnp.exp(logits - m), axis=1, keepdims=True)

    onehot = (iota_e == top1).astype(jnp.int32)
    inc = onehot
    k = 1
    while k < T:
        shifted = jnp.concatenate(
            [jnp.zeros((k, E), jnp.int32), inc[: T - k, :]], axis=0)
        inc = inc + shifted
        k *= 2
    rank = jnp.sum(onehot * inc, axis=1, keepdims=True) - 1
    counts = inc[T - 1:T, :]

    ntiles = (counts + BT - 1) // BT
    cum = ntiles
    k = 1
    while k < E:
        shifted = jnp.concatenate(
            [jnp.zeros((1, k), jnp.int32), cum[:, : E - k]], axis=1)
        cum = cum + shifted
        k *= 2
    excl = cum - ntiles
    start_tok = jnp.sum(onehot * (BT * excl), axis=1, keepdims=True)
    pos_ref[...] = (start_tok + rank).reshape(T)

    x_bf = x.astype(jnp.bfloat16)
    lo = lax.bitcast_convert_type(x_bf[:, :XH], jnp.uint16).astype(jnp.uint32)
    hi = lax.bitcast_convert_type(x_bf[:, XH:], jnp.uint16).astype(jnp.uint32)
    xi = lo | (hi << 16)
    pb = lax.bitcast_convert_type(pmax.astype(jnp.bfloat16),
                                  jnp.uint16).astype(jnp.uint32)
    pi = jnp.broadcast_to(pb | (pb << 16), (T, XW - XH))
    pw_ref[...] = lax.bitcast_convert_type(
        jnp.concatenate([xi, pi], axis=1), jnp.int32)

    ii = lax.broadcasted_iota(jnp.int32, (NT, E), 0)
    cum_b = jnp.broadcast_to(cum, (NT, E))
    eid = jnp.sum((cum_b <= ii).astype(jnp.int32), axis=1, keepdims=True)
    iota_e_row = lax.broadcasted_iota(jnp.int32, (1, E), 1)
    last_e = jnp.max(jnp.where(counts > 0, iota_e_row, 0))
    eid_ref[...] = jnp.minimum(eid, last_e).reshape(NT)
    total = cum[0, E - 1]
    ii1 = lax.broadcasted_iota(jnp.int32, (NT, 1), 0)
    blk_ref[...] = jnp.where(ii1 < total, ii1, total - 1).reshape(NT)


def _run_router(xf, Wg, bg):
    return pl.pallas_call(
        _router_body,
        out_shape=[
            jax.ShapeDtypeStruct((T,), jnp.int32),
            jax.ShapeDtypeStruct((T, XW), jnp.int32),
            jax.ShapeDtypeStruct((NT,), jnp.int32),
            jax.ShapeDtypeStruct((NT,), jnp.int32),
        ],
    )(xf, Wg, bg.reshape(1, E))


def _dispatch_body(pos_hbm, pw_hbm, xs_hbm, idx_v, rows_v, sem_x):
    wid = lax.axis_index("s") * NC + lax.axis_index("c")
    base = wid * PER_W
    pltpu.sync_copy(pos_hbm.at[pl.ds(base, PER_W)], idx_v)
    pltpu.sync_copy(pw_hbm.at[pl.ds(base, PER_W)], rows_v)
    pltpu.async_copy(rows_v, xs_hbm.at[idx_v], sem_x).wait()


def _run_dispatch(pos, pw):
    f = functools.partial(
        pl.kernel,
        out_type=jax.ShapeDtypeStruct((XS_PAD, XW), jnp.int32),
        mesh=plsc.VectorSubcoreMesh(core_axis_name="c", subcore_axis_name="s"),
        scratch_types=[
            pltpu.VMEM((PER_W,), jnp.int32),
            pltpu.VMEM((PER_W, XW), jnp.int32),
            pltpu.SemaphoreType.DMA,
        ],
    )(_dispatch_body)
    return f(pos, pw)


def _ffn_body(eids, blks, xs_ref, w1_ref, b1_ref, w2_ref, b2_ref, os_ref):
    u = lax.bitcast_convert_type(xs_ref[...], jnp.uint32)
    xi = u[:, :XH]
    xlo = lax.bitcast_convert_type((xi & 0xFFFF).astype(jnp.uint16),
                                   jnp.bfloat16).astype(jnp.float32)
    xhi = lax.bitcast_convert_type((xi >> 16).astype(jnp.uint16),
                                   jnp.bfloat16).astype(jnp.float32)
    x = jnp.concatenate([xlo, xhi], axis=1)
    p = lax.bitcast_convert_type(
        (u[:, XH:XH + 1] & 0xFFFF).astype(jnp.uint16),
        jnp.bfloat16).astype(jnp.float32)
    h = jnp.dot(x, w1_ref[0], preferred_element_type=jnp.float32)
    h = h + b1_ref[0]
    h = 0.5 * h * (1.0 + lax.erf(h * (1.0 / math.sqrt(2.0))))
    o = jnp.dot(h, w2_ref[0], preferred_element_type=jnp.float32)
    o = o + b2_ref[0]
    os_ref[...] = o * p


def _run_ffn(eids, blks, xs, W1, b1, W2, b2):
    grid_spec = pltpu.PrefetchScalarGridSpec(
        num_scalar_prefetch=2,
        grid=(NT,),
        in_specs=[
            pl.BlockSpec((BT, XW), lambda i, eids, blks: (blks[i], 0)),
            pl.BlockSpec((1, D, H), lambda i, eids, blks: (eids[i], 0, 0)),
            pl.BlockSpec((1, 1, H), lambda i, eids, blks: (eids[i], 0, 0)),
            pl.BlockSpec((1, H, D), lambda i, eids, blks: (eids[i], 0, 0)),
            pl.BlockSpec((1, 1, D), lambda i, eids, blks: (eids[i], 0, 0)),
        ],
        out_specs=pl.BlockSpec((BT, D), lambda i, eids, blks: (blks[i], 0)),
    )
    return pl.pallas_call(
        _ffn_body,
        grid_spec=grid_spec,
        out_shape=jax.ShapeDtypeStruct((XS_PAD, D), jnp.float32),
    )(eids, blks, xs, W1, b1.reshape(E, 1, H), W2, b2.reshape(E, 1, D))


def _combine_body(pos_hbm, os_hbm, out_hbm,
                  idx0, idx1, rows0, rows1, sem0, sem1):
    wid = lax.axis_index("s") * NC + lax.axis_index("c")
    base = wid * PER_W
    pltpu.sync_copy(pos_hbm.at[pl.ds(base, CH)], idx0)
    g0 = pltpu.async_copy(os_hbm.at[idx0], rows0, sem0)
    pltpu.sync_copy(pos_hbm.at[pl.ds(base + CH, CH)], idx1)
    g1 = pltpu.async_copy(os_hbm.at[idx1], rows1, sem1)
    g0.wait()
    pltpu.sync_copy(rows0, out_hbm.at[pl.ds(base, CH)])
    g1.wait()
    pltpu.sync_copy(rows1, out_hbm.at[pl.ds(base + CH, CH)])


def _run_combine(pos, os):
    f = functools.partial(
        pl.kernel,
        out_type=jax.ShapeDtypeStruct((T, D), jnp.float32),
        mesh=plsc.VectorSubcoreMesh(core_axis_name="c", subcore_axis_name="s"),
        scratch_types=[
            pltpu.VMEM((CH,), jnp.int32),
            pltpu.VMEM((CH,), jnp.int32),
            pltpu.VMEM((CH, D), jnp.float32),
            pltpu.VMEM((CH, D), jnp.float32),
            pltpu.SemaphoreType.DMA,
            pltpu.SemaphoreType.DMA,
        ],
    )(_combine_body)
    return f(pos, os)


@jax.jit
def kernel(x, W1, b1, W2, b2, Wg, bg):
    xf = x.reshape(T, D)
    pos, pw, eids, blks = _run_router(xf, Wg, bg)
    xs = _run_dispatch(pos, pw)
    os = _run_ffn(eids, blks, xs, W1, b1, W2, b2)
    out = _run_combine(pos, os)
    return out.reshape(B, S, D)


# speedup vs baseline: 1.0251x; 1.0089x over previous
import functools
import math

import jax
import jax.numpy as jnp
from jax import lax
from jax.experimental import pallas as pl
from jax.experimental.pallas import tpu as pltpu
from jax.experimental.pallas import tpu_sc as plsc

B, S, D, H, E = 2, 2048, 768, 1536, 64
T = B * S
BT = 128
NT = 96
XS_PAD = NT * BT
XH = D // 2
XW = XH + 128

NC, NS = 2, 16
NW = NC * NS
PER_W = T // NW
CH = 64


def _router_body(x_ref, wg_ref, bg_ref, pos_ref, pw_ref, eid_ref, blk_ref):
    x = x_ref[...].reshape(T, D)
    logits = jnp.dot(x, wg_ref[...], preferred_element_type=jnp.float32)
    logits = logits + bg_ref[...]
    m = jnp.max(logits, axis=1, keepdims=True)
    iota_e = lax.broadcasted_iota(jnp.int32, (T, E), 1)
    top1 = jnp.min(jnp.where(logits == m, iota_e, E), axis=1, keepdims=True)
    pmax = 1.0 / jnp.sum(jnp.exp(logits - m), axis=1, keepdims=True)

    onehot = (iota_e == top1).astype(jnp.int32)
    inc = onehot
    k = 1
    while k < T:
        shifted = jnp.concatenate(
            [jnp.zeros((k, E), jnp.int32), inc[: T - k, :]], axis=0)
        inc = inc + shifted
        k *= 2
    counts = inc[T - 1:T, :]

    ntiles = (counts + BT - 1) // BT
    cum = ntiles
    k = 1
    while k < E:
        shifted = jnp.concatenate(
            [jnp.zeros((1, k), jnp.int32), cum[:, : E - k]], axis=1)
        cum = cum + shifted
        k *= 2
    excl = cum - ntiles
    pos = jnp.sum(jnp.where(onehot != 0, inc + BT * excl, 0), axis=1) - 1
    pos_ref[...] = pos

    x_bf = x.astype(jnp.bfloat16)
    lo = lax.bitcast_convert_type(x_bf[:, :XH], jnp.uint16).astype(jnp.uint32)
    hi = lax.bitcast_convert_type(x_bf[:, XH:], jnp.uint16).astype(jnp.uint32)
    xi = lo | (hi << 16)
    pb = lax.bitcast_convert_type(pmax.astype(jnp.bfloat16),
                                  jnp.uint16).astype(jnp.uint32)
    pi = jnp.broadcast_to(pb | (pb << 16), (T, XW - XH))
    pw_ref[...] = lax.bitcast_convert_type(
        jnp.concatenate([xi, pi], axis=1), jnp.int32)

    ii = lax.broadcasted_iota(jnp.int32, (NT, E), 0)
    cum_b = jnp.broadcast_to(cum, (NT, E))
    eid = jnp.sum((cum_b <= ii).astype(jnp.int32), axis=1, keepdims=True)
    iota_e_row = lax.broadcasted_iota(jnp.int32, (1, E), 1)
    last_e = jnp.max(jnp.where(counts > 0, iota_e_row, 0))
    eid_ref[...] = jnp.minimum(eid, last_e).reshape(NT)
    total = cum[0, E - 1]
    ii1 = lax.broadcasted_iota(jnp.int32, (NT, 1), 0)
    blk_ref[...] = jnp.where(ii1 < total, ii1, total - 1).reshape(NT)


def _run_router(x, Wg, bg):
    return pl.pallas_call(
        _router_body,
        out_shape=[
            jax.ShapeDtypeStruct((T,), jnp.int32),
            jax.ShapeDtypeStruct((T, XW), jnp.int32),
            jax.ShapeDtypeStruct((NT,), jnp.int32),
            jax.ShapeDtypeStruct((NT,), jnp.int32),
        ],
    )(x, Wg, bg.reshape(1, E))


def _dispatch_body(pos_hbm, pw_hbm, xs_hbm, idx_v, rows_v, sem_x):
    wid = lax.axis_index("s") * NC + lax.axis_index("c")
    base = wid * PER_W
    pltpu.sync_copy(pos_hbm.at[pl.ds(base, PER_W)], idx_v)
    pltpu.sync_copy(pw_hbm.at[pl.ds(base, PER_W)], rows_v)
    pltpu.async_copy(rows_v, xs_hbm.at[idx_v], sem_x).wait()


def _run_dispatch(pos, pw):
    f = functools.partial(
        pl.kernel,
        out_type=jax.ShapeDtypeStruct((XS_PAD, XW), jnp.int32),
        mesh=plsc.VectorSubcoreMesh(core_axis_name="c", subcore_axis_name="s"),
        scratch_types=[
            pltpu.VMEM((PER_W,), jnp.int32),
            pltpu.VMEM((PER_W, XW), jnp.int32),
            pltpu.SemaphoreType.DMA,
        ],
    )(_dispatch_body)
    return f(pos, pw)


def _ffn_body(eids, blks, xs_ref, w1_ref, b1_ref, w2_ref, b2_ref, os_ref):
    u = lax.bitcast_convert_type(xs_ref[...], jnp.uint32)
    xi = u[:, :XH]
    xlo = lax.bitcast_convert_type((xi & 0xFFFF).astype(jnp.uint16),
                                   jnp.bfloat16).astype(jnp.float32)
    xhi = lax.bitcast_convert_type((xi >> 16).astype(jnp.uint16),
                                   jnp.bfloat16).astype(jnp.float32)
    x = jnp.concatenate([xlo, xhi], axis=1)
    p = lax.bitcast_convert_type(
        (u[:, XH:XH + 1] & 0xFFFF).astype(jnp.uint16),
        jnp.bfloat16).astype(jnp.float32)
    h = jnp.dot(x, w1_ref[0], preferred_element_type=jnp.float32)
    h = h + b1_ref[0]
    h = 0.5 * h * (1.0 + lax.erf(h * (1.0 / math.sqrt(2.0))))
    o = jnp.dot(h, w2_ref[0], preferred_element_type=jnp.float32)
    o = o + b2_ref[0]
    os_ref[...] = o * p


def _run_ffn(eids, blks, xs, W1, b1, W2, b2):
    grid_spec = pltpu.PrefetchScalarGridSpec(
        num_scalar_prefetch=2,
        grid=(NT,),
        in_specs=[
            pl.BlockSpec((BT, XW), lambda i, eids, blks: (blks[i], 0)),
            pl.BlockSpec((1, D, H), lambda i, eids, blks: (eids[i], 0, 0)),
            pl.BlockSpec((1, 1, H), lambda i, eids, blks: (eids[i], 0, 0)),
            pl.BlockSpec((1, H, D), lambda i, eids, blks: (eids[i], 0, 0)),
            pl.BlockSpec((1, 1, D), lambda i, eids, blks: (eids[i], 0, 0)),
        ],
        out_specs=pl.BlockSpec((BT, D), lambda i, eids, blks: (blks[i], 0)),
    )
    return pl.pallas_call(
        _ffn_body,
        grid_spec=grid_spec,
        out_shape=jax.ShapeDtypeStruct((XS_PAD, D), jnp.float32),
    )(eids, blks, xs, W1, b1.reshape(E, 1, H), W2, b2.reshape(E, 1, D))


def _combine_body(pos_hbm, os_hbm, out_hbm,
                  idx0, idx1, rows0, rows1, sem0, sem1):
    wid = lax.axis_index("s") * NC + lax.axis_index("c")
    base = wid * PER_W
    pltpu.sync_copy(pos_hbm.at[pl.ds(base, CH)], idx0)
    g0 = pltpu.async_copy(os_hbm.at[idx0], rows0, sem0)
    pltpu.sync_copy(pos_hbm.at[pl.ds(base + CH, CH)], idx1)
    g1 = pltpu.async_copy(os_hbm.at[idx1], rows1, sem1)
    g0.wait()
    pltpu.sync_copy(rows0, out_hbm.at[pl.ds(base, CH)])
    g1.wait()
    pltpu.sync_copy(rows1, out_hbm.at[pl.ds(base + CH, CH)])


def _run_combine(pos, os):
    f = functools.partial(
        pl.kernel,
        out_type=jax.ShapeDtypeStruct((T, D), jnp.float32),
        mesh=plsc.VectorSubcoreMesh(core_axis_name="c", subcore_axis_name="s"),
        scratch_types=[
            pltpu.VMEM((CH,), jnp.int32),
            pltpu.VMEM((CH,), jnp.int32),
            pltpu.VMEM((CH, D), jnp.float32),
            pltpu.VMEM((CH, D), jnp.float32),
            pltpu.SemaphoreType.DMA,
            pltpu.SemaphoreType.DMA,
        ],
    )(_combine_body)
    return f(pos, os)


@jax.jit
def kernel(x, W1, b1, W2, b2, Wg, bg):
    pos, pw, eids, blks = _run_router(x, Wg, bg)
    xs = _run_dispatch(pos, pw)
    os = _run_ffn(eids, blks, xs, W1, b1, W2, b2)
    out = _run_combine(pos, os)
    return out.reshape(B, S, D)


# speedup vs baseline: 1.0335x; 1.0082x over previous
import functools
import math

import jax
import jax.numpy as jnp
from jax import lax
from jax.experimental import pallas as pl
from jax.experimental.pallas import tpu as pltpu
from jax.experimental.pallas import tpu_sc as plsc

B, S, D, H, E = 2, 2048, 768, 1536, 64
T = B * S
BT = 128
NT = 96
XS_PAD = NT * BT
XH = D // 2
XW = XH + 128

NC, NS = 2, 16
NW = NC * NS
PER_W = T // NW
CH = 64


def _router_body(x_ref, wg_ref, bg_ref, pos_ref, pw_ref, eb_ref):
    x = x_ref[...].reshape(T, D)
    logits = jnp.dot(x, wg_ref[...], preferred_element_type=jnp.float32)
    logits = logits + bg_ref[...]
    m = jnp.max(logits, axis=1, keepdims=True)
    iota_e = lax.broadcasted_iota(jnp.int32, (T, E), 1)
    top1 = jnp.min(jnp.where(logits == m, iota_e, E), axis=1, keepdims=True)
    pmax = 1.0 / jnp.sum(jnp.exp(logits - m), axis=1, keepdims=True)

    onehot = (iota_e == top1).astype(jnp.int32)
    inc = onehot
    k = 1
    while k < T:
        shifted = jnp.concatenate(
            [jnp.zeros((k, E), jnp.int32), inc[: T - k, :]], axis=0)
        inc = inc + shifted
        k *= 2
    counts = inc[T - 1:T, :]

    ntiles = (counts + BT - 1) // BT
    cum = ntiles
    k = 1
    while k < E:
        shifted = jnp.concatenate(
            [jnp.zeros((1, k), jnp.int32), cum[:, : E - k]], axis=1)
        cum = cum + shifted
        k *= 2
    excl = cum - ntiles
    pos = jnp.sum(jnp.where(onehot != 0, inc + BT * excl, 0), axis=1) - 1
    pos_ref[...] = pos

    x_bf = x.astype(jnp.bfloat16)
    lo = lax.bitcast_convert_type(x_bf[:, :XH], jnp.uint16).astype(jnp.uint32)
    hi = lax.bitcast_convert_type(x_bf[:, XH:], jnp.uint16).astype(jnp.uint32)
    xi = lo | (hi << 16)
    pb = lax.bitcast_convert_type(pmax.astype(jnp.bfloat16),
                                  jnp.uint16).astype(jnp.uint32)
    pi = jnp.broadcast_to(pb | (pb << 16), (T, XW - XH))
    pw_ref[...] = lax.bitcast_convert_type(
        jnp.concatenate([xi, pi], axis=1), jnp.int32)

    ii = lax.broadcasted_iota(jnp.int32, (NT, E), 0)
    cum_b = jnp.broadcast_to(cum, (NT, E))
    eid = jnp.sum((cum_b <= ii).astype(jnp.int32), axis=1, keepdims=True)
    iota_e_row = lax.broadcasted_iota(jnp.int32, (1, E), 1)
    last_e = jnp.max(jnp.where(counts > 0, iota_e_row, 0))
    eids = jnp.minimum(eid, last_e)
    total = cum[0, E - 1]
    ii1 = lax.broadcasted_iota(jnp.int32, (NT, 1), 0)
    blks = jnp.where(ii1 < total, ii1, total - 1)
    eb_ref[...] = jnp.concatenate([eids, blks], axis=0).reshape(2 * NT)


def _run_router(x, Wg, bg):
    return pl.pallas_call(
        _router_body,
        out_shape=[
            jax.ShapeDtypeStruct((T,), jnp.int32),
            jax.ShapeDtypeStruct((T, XW), jnp.int32),
            jax.ShapeDtypeStruct((2 * NT,), jnp.int32),
        ],
    )(x, Wg, bg.reshape(1, E))


def _dispatch_body(pos_hbm, pw_hbm, xs_hbm, idx_v, rows_v, sem_x):
    wid = lax.axis_index("s") * NC + lax.axis_index("c")
    base = wid * PER_W
    pltpu.sync_copy(pos_hbm.at[pl.ds(base, PER_W)], idx_v)
    pltpu.sync_copy(pw_hbm.at[pl.ds(base, PER_W)], rows_v)
    pltpu.async_copy(rows_v, xs_hbm.at[idx_v], sem_x).wait()


def _run_dispatch(pos, pw):
    f = functools.partial(
        pl.kernel,
        out_type=jax.ShapeDtypeStruct((XS_PAD, XW), jnp.int32),
        mesh=plsc.VectorSubcoreMesh(core_axis_name="c", subcore_axis_name="s"),
        scratch_types=[
            pltpu.VMEM((PER_W,), jnp.int32),
            pltpu.VMEM((PER_W, XW), jnp.int32),
            pltpu.SemaphoreType.DMA,
        ],
    )(_dispatch_body)
    return f(pos, pw)


def _ffn_body(eb, xs_ref, w1_ref, b1_ref, w2_ref, b2_ref, os_ref):
    u = lax.bitcast_convert_type(xs_ref[...], jnp.uint32)
    xi = u[:, :XH]
    xlo = lax.bitcast_convert_type((xi & 0xFFFF).astype(jnp.uint16),
                                   jnp.bfloat16).astype(jnp.float32)
    xhi = lax.bitcast_convert_type((xi >> 16).astype(jnp.uint16),
                                   jnp.bfloat16).astype(jnp.float32)
    x = jnp.concatenate([xlo, xhi], axis=1)
    p = lax.bitcast_convert_type(
        (u[:, XH:XH + 1] & 0xFFFF).astype(jnp.uint16),
        jnp.bfloat16).astype(jnp.float32)
    h = jnp.dot(x, w1_ref[0], preferred_element_type=jnp.float32)
    h = h + b1_ref[0]
    h = 0.5 * h * (1.0 + lax.erf(h * (1.0 / math.sqrt(2.0))))
    o = jnp.dot(h, w2_ref[0], preferred_element_type=jnp.float32)
    o = o + b2_ref[0]
    os_ref[...] = o * p


def _run_ffn(eb, xs, W1, b1, W2, b2):
    grid_spec = pltpu.PrefetchScalarGridSpec(
        num_scalar_prefetch=1,
        grid=(NT,),
        in_specs=[
            pl.BlockSpec((BT, XW), lambda i, eb: (eb[NT + i], 0)),
            pl.BlockSpec((1, D, H), lambda i, eb: (eb[i], 0, 0)),
            pl.BlockSpec((1, 1, H), lambda i, eb: (eb[i], 0, 0)),
            pl.BlockSpec((1, H, D), lambda i, eb: (eb[i], 0, 0)),
            pl.BlockSpec((1, 1, D), lambda i, eb: (eb[i], 0, 0)),
        ],
        out_specs=pl.BlockSpec((BT, D), lambda i, eb: (eb[NT + i], 0)),
    )
    return pl.pallas_call(
        _ffn_body,
        grid_spec=grid_spec,
        out_shape=jax.ShapeDtypeStruct((XS_PAD, D), jnp.float32),
    )(eb, xs, W1, b1.reshape(E, 1, H), W2, b2.reshape(E, 1, D))


def _combine_body(pos_hbm, os_hbm, out_hbm,
                  idx0, idx1, rows0, rows1, sem0, sem1):
    wid = lax.axis_index("s") * NC + lax.axis_index("c")
    base = wid * PER_W
    pltpu.sync_copy(pos_hbm.at[pl.ds(base, CH)], idx0)
    g0 = pltpu.async_copy(os_hbm.at[idx0], rows0, sem0)
    pltpu.sync_copy(pos_hbm.at[pl.ds(base + CH, CH)], idx1)
    g1 = pltpu.async_copy(os_hbm.at[idx1], rows1, sem1)
    g0.wait()
    pltpu.sync_copy(rows0, out_hbm.at[pl.ds(base, CH)])
    g1.wait()
    pltpu.sync_copy(rows1, out_hbm.at[pl.ds(base + CH, CH)])


def _run_combine(pos, os):
    f = functools.partial(
        pl.kernel,
        out_type=jax.ShapeDtypeStruct((T, D), jnp.float32),
        mesh=plsc.VectorSubcoreMesh(core_axis_name="c", subcore_axis_name="s"),
        scratch_types=[
            pltpu.VMEM((CH,), jnp.int32),
            pltpu.VMEM((CH,), jnp.int32),
            pltpu.VMEM((CH, D), jnp.float32),
            pltpu.VMEM((CH, D), jnp.float32),
            pltpu.SemaphoreType.DMA,
            pltpu.SemaphoreType.DMA,
        ],
    )(_combine_body)
    return f(pos, os)


@jax.jit
def kernel(x, W1, b1, W2, b2, Wg, bg):
    pos, pw, eb = _run_router(x, Wg, bg)
    xs = _run_dispatch(pos, pw)
    os = _run_ffn(eb, xs, W1, b1, W2, b2)
    out = _run_combine(pos, os)
    return out.reshape(B, S, D)
